# baseline (speedup 1.0000x reference)
import functools

import jax
import jax.numpy as jnp
from jax import lax
from jax.experimental import pallas as pl
from jax.experimental.pallas import tpu as pltpu, tpu_sc as plsc

B, T_PH, T_W = 16, 2048, 1024
EPS = 1e-05
L = 16
H = T_PH // 2
MAGIC = 12582912.0


def _body(ph_hbm, idx_hbm, wd_hbm, zero_hbm, out_hbm,
          ph_v, idx_v, wd_v, seg_v, out_v, sem):
    row = lax.axis_index("s")
    half = lax.axis_index("c")
    base = half * H

    cp_ph = pltpu.async_copy(ph_hbm.at[row], ph_v, sem)
    cp_ix = pltpu.async_copy(idx_hbm.at[row], idx_v, sem)
    cp_wd = pltpu.async_copy(wd_hbm.at[row], wd_v, sem)
    cp_z = pltpu.async_copy(zero_hbm, seg_v, sem)
    cp_ph.wait()
    cp_ix.wait()
    cp_wd.wait()
    cp_z.wait()

    def scat_step(i, _):
        idx = idx_v[pl.ds(i * L, L)]
        vals = ph_v[pl.ds(i * L, L)]
        mask = idx > 0
        plsc.addupdate_scatter(seg_v, [jnp.maximum(idx - 1, 0)], vals,
                               mask=mask)
        return 0

    lax.fori_loop(0, T_PH // L, scat_step, 0, unroll=4)

    def gath_step(i, _):
        off = base + i * L
        idx = idx_v[pl.ds(off, L)]
        vals = ph_v[pl.ds(off, L)]
        mask = idx > 0
        gi = jnp.maximum(idx - 1, 0)
        s = plsc.load_gather(seg_v, [gi], mask=mask)
        w = plsc.load_gather(wd_v, [gi], mask=mask)
        a = w / jnp.maximum(s, EPS)
        x = jnp.where(mask, vals * a, 0.0)
        r = (x + MAGIC) - MAGIC
        out_v[pl.ds(i * L, L)] = r.astype(jnp.int32)
        return 0

    lax.fori_loop(0, H // L, gath_step, 0, unroll=4)

    pltpu.sync_copy(out_v, out_hbm.at[row, pl.ds(base, H)])


@jax.jit
def _regulate(ph_dur, ph2word_i32, word_dur):
    mesh = plsc.VectorSubcoreMesh(core_axis_name="c", subcore_axis_name="s")
    f = functools.partial(
        pl.kernel,
        out_type=jax.ShapeDtypeStruct((B, T_PH), jnp.int32),
        mesh=mesh,
        compiler_params=pltpu.CompilerParams(needs_layout_passes=False),
        scratch_types=[
            pltpu.VMEM((T_PH,), jnp.float32),
            pltpu.VMEM((T_PH,), jnp.int32),
            pltpu.VMEM((T_W,), jnp.float32),
            pltpu.VMEM((T_W,), jnp.float32),
            pltpu.VMEM((H,), jnp.int32),
            pltpu.SemaphoreType.DMA,
        ],
    )(_body)
    zeros = jnp.zeros((T_W,), jnp.float32)
    return f(ph_dur, ph2word_i32, word_dur, zeros)


def kernel(ph_dur, ph2word, word_dur):
    out = _regulate(ph_dur.astype(jnp.float32), ph2word.astype(jnp.int32),
                    word_dur.astype(jnp.float32))
    return out.astype(jnp.int64)


# speedup vs baseline: 1.1433x; 1.1433x over previous
import functools

import jax
import jax.numpy as jnp
from jax import lax
from jax.experimental import pallas as pl
from jax.experimental.pallas import tpu as pltpu, tpu_sc as plsc

B, T_PH, T_W = 16, 2048, 1024
EPS = 1e-05
L = 16
H = T_PH // 2
MAGIC = 12582912.0


def _body(ph_hbm, idx_hbm, wd_hbm, out_hbm,
          ph_v, idx_v, wd_v, seg_v, out_v, sem_ph, sem_ix, sem_wd):
    row = lax.axis_index("s")
    half = lax.axis_index("c")
    base = half * H

    cp_ph = pltpu.async_copy(ph_hbm.at[row], ph_v, sem_ph)
    cp_ix = pltpu.async_copy(idx_hbm.at[row], idx_v, sem_ix)
    cp_wd = pltpu.async_copy(wd_hbm.at[row], wd_v, sem_wd)

    zeros = jnp.zeros((L,), jnp.float32)

    @plsc.parallel_loop(0, T_W // L, unroll=8)
    def _(i):
        seg_v[pl.ds(i * L, L)] = zeros

    cp_ix.wait()
    cp_ph.wait()

    @plsc.parallel_loop(0, T_PH // L, unroll=4)
    def _(i):
        idx = idx_v[pl.ds(i * L, L)]
        vals = ph_v[pl.ds(i * L, L)]
        mask = idx > 0
        plsc.addupdate_scatter(seg_v, [jnp.maximum(idx - 1, 0)], vals,
                               mask=mask)

    cp_wd.wait()

    @plsc.parallel_loop(0, H // L, unroll=4)
    def _(i):
        off = base + i * L
        idx = idx_v[pl.ds(off, L)]
        vals = ph_v[pl.ds(off, L)]
        mask = idx > 0
        gi = jnp.maximum(idx - 1, 0)
        s = plsc.load_gather(seg_v, [gi], mask=mask)
        w = plsc.load_gather(wd_v, [gi], mask=mask)
        a = w / jnp.maximum(s, EPS)
        x = jnp.where(mask, vals * a, 0.0)
        r = (x + MAGIC) - MAGIC
        out_v[pl.ds(i * L, L)] = r.astype(jnp.int32)

    pltpu.sync_copy(out_v, out_hbm.at[row, pl.ds(base, H)])


@jax.jit
def _regulate(ph_dur, ph2word_i32, word_dur):
    mesh = plsc.VectorSubcoreMesh(core_axis_name="c", subcore_axis_name="s")
    f = functools.partial(
        pl.kernel,
        out_type=jax.ShapeDtypeStruct((B, T_PH), jnp.int32),
        mesh=mesh,
        compiler_params=pltpu.CompilerParams(needs_layout_passes=False),
        scratch_types=[
            pltpu.VMEM((T_PH,), jnp.float32),
            pltpu.VMEM((T_PH,), jnp.int32),
            pltpu.VMEM((T_W,), jnp.float32),
            pltpu.VMEM((T_W,), jnp.float32),
            pltpu.VMEM((H,), jnp.int32),
            pltpu.SemaphoreType.DMA,
            pltpu.SemaphoreType.DMA,
            pltpu.SemaphoreType.DMA,
        ],
    )(_body)
    return f(ph_dur, ph2word_i32, word_dur)


def kernel(ph_dur, ph2word, word_dur):
    out = _regulate(ph_dur.astype(jnp.float32), ph2word.astype(jnp.int32),
                    word_dur.astype(jnp.float32))
    return out.astype(jnp.int64)
